# baseline (speedup 1.0000x reference)
import functools

import jax
import jax.numpy as jnp
from jax.experimental import pallas as pl
from jax.experimental.pallas import tpu as pltpu
from jax.experimental.pallas import tpu_sc as plsc

E = 64
D = 768
FF = 256
T = 2048
BT = 64
NW = 32
TOK = T // NW
OFFS_LEN = 80


def _router_body(x_ref, gw_ref, logits_ref, pos_ref, wtop_ref, offs_ref):
    x = x_ref[...]
    gw = gw_ref[...]
    logits = jax.lax.dot_general(
        x, gw, (((1,), (1,)), ((), ())), preferred_element_type=jnp.float32)
    logits_ref[...] = logits

    m = jnp.max(logits, axis=1, keepdims=True)
    s = jnp.sum(jnp.exp(logits - m), axis=1, keepdims=True)
    wtop_ref[...] = jnp.broadcast_to(1.0 / s, (T, 128))

    iota_e = jax.lax.broadcasted_iota(jnp.int32, (T, E), 1)
    eidx = jnp.min(jnp.where(logits == m, iota_e, E), axis=1, keepdims=True)
    onehot = (iota_e == eidx).astype(jnp.bfloat16)

    counts = jnp.sum(onehot.astype(jnp.float32), axis=0, keepdims=True)
    bi = jax.lax.broadcasted_iota(jnp.int32, (E, E), 0)
    bj = jax.lax.broadcasted_iota(jnp.int32, (E, E), 1)
    lt = (bi < bj).astype(jnp.float32)
    offs = jax.lax.dot_general(
        counts, lt, (((1,), (0,)), ((), ())), preferred_element_type=jnp.float32)

    blk = 256
    csum_parts = []
    for k in range(T // blk):
        ri = jax.lax.broadcasted_iota(jnp.int32, (blk, T), 0) + k * blk
        ci = jax.lax.broadcasted_iota(jnp.int32, (blk, T), 1)
        g = (ci < ri).astype(jnp.bfloat16)
        csum_parts.append(jax.lax.dot_general(
            g, onehot, (((1,), (0,)), ((), ())),
            preferred_element_type=jnp.float32))
    csum = jnp.concatenate(csum_parts, axis=0)

    posf = jnp.sum((csum + offs) * onehot.astype(jnp.float32),
                   axis=1, keepdims=True)
    pos_ref[...] = posf.astype(jnp.int32)

    offs_i = offs.astype(jnp.int32)
    offs_ref[...] = jnp.concatenate(
        [offs_i, jnp.full((1, OFFS_LEN - E), T, jnp.int32)], axis=1)


def _router(x, gate_w):
    return pl.pallas_call(
        _router_body,
        out_shape=(
            jax.ShapeDtypeStruct((T, E), jnp.float32),
            jax.ShapeDtypeStruct((T, 1), jnp.int32),
            jax.ShapeDtypeStruct((T, 128), jnp.float32),
            jax.ShapeDtypeStruct((1, OFFS_LEN), jnp.int32),
        ),
        compiler_params=pltpu.CompilerParams(vmem_limit_bytes=100_000_000),
    )(x, gate_w)


def _dispatch_body(x_hbm, pos_hbm, w16_hbm, xs_hbm, ws_hbm,
                   pos_v, rows_v, w16_v, sem_x, sem_w):
    wid = jax.lax.axis_index("s") * 2 + jax.lax.axis_index("c")
    base = wid * TOK
    pltpu.sync_copy(pos_hbm.at[pl.ds(base, TOK)], pos_v)
    pltpu.sync_copy(x_hbm.at[pl.ds(base, TOK)], rows_v)
    pltpu.sync_copy(w16_hbm.at[pl.ds(base, TOK)], w16_v)
    cp_x = pltpu.async_copy(rows_v, xs_hbm.at[pos_v], sem_x)
    cp_w = pltpu.async_copy(w16_v, ws_hbm.at[pos_v], sem_w)
    cp_x.wait()
    cp_w.wait()


@functools.cache
def _sc_kernels():
    mesh = plsc.VectorSubcoreMesh(core_axis_name="c", subcore_axis_name="s")
    dispatch = pl.kernel(
        _dispatch_body,
        out_type=[
            jax.ShapeDtypeStruct((T, D), jnp.float32),
            jax.ShapeDtypeStruct((T, 128), jnp.float32),
        ],
        mesh=mesh,
        scratch_types=[
            pltpu.VMEM((TOK,), jnp.int32),
            pltpu.VMEM((TOK, D), jnp.float32),
            pltpu.VMEM((TOK, 128), jnp.float32),
            pltpu.SemaphoreType.DMA,
            pltpu.SemaphoreType.DMA,
        ],
    )
    combine = pl.kernel(
        _combine_body,
        out_type=jax.ShapeDtypeStruct((T, D), jnp.float32),
        mesh=mesh,
        scratch_types=[
            pltpu.VMEM((TOK,), jnp.int32),
            pltpu.VMEM((TOK, D), jnp.float32),
            pltpu.SemaphoreType.DMA,
        ],
    )
    return dispatch, combine


EPG = 8


def _mlp_body(offs_ref, xs_ref, ws_ref, wg_ref, wu_ref, wd_ref, ys_ref):
    step = pl.program_id(0)

    @pl.when(step == 0)
    def _():
        ys_ref[...] = jnp.zeros_like(ys_ref)

    for j in range(EPG):
        e = step * EPG + j
        start = offs_ref[0, e]
        end = offs_ref[0, e + 1]
        s8 = (start // 8) * 8
        nb = (end - s8 + BT - 1) // BT
        wg = wg_ref[j].astype(jnp.bfloat16)
        wu = wu_ref[j].astype(jnp.bfloat16)
        wd = wd_ref[j].astype(jnp.bfloat16)

        def body(b, _, start=start, end=end, s8=s8, wg=wg, wu=wu, wd=wd):
            row0 = s8 + b * BT
            row0c = pl.multiple_of(jnp.minimum(row0, T - BT), 8)
            xb = xs_ref[pl.ds(row0c, BT), :].astype(jnp.bfloat16)
            g = jax.lax.dot_general(
                xb, wg, (((1,), (1,)), ((), ())),
                preferred_element_type=jnp.float32)
            u = jax.lax.dot_general(
                xb, wu, (((1,), (1,)), ((), ())),
                preferred_element_type=jnp.float32)
            h = g * (1.0 / (1.0 + jnp.exp(-g))) * u
            y = jax.lax.dot_general(
                h.astype(jnp.bfloat16), wd, (((1,), (1,)), ((), ())),
                preferred_element_type=jnp.float32)
            rows = row0c + jax.lax.broadcasted_iota(jnp.int32, (BT, 1), 0)
            mask = ((rows >= jnp.maximum(row0, start))
                    & (rows < jnp.minimum(row0 + BT, end))).astype(jnp.float32)
            wv = ws_ref[pl.ds(row0c, BT), 0:1]
            ys_ref[pl.ds(row0c, BT), :] += y * (wv * mask)
            return 0

        jax.lax.fori_loop(0, nb, body, 0)


def _mlp(offs, xs, ws, w_gate_proj, w_up_proj, w_down_proj):
    return pl.pallas_call(
        _mlp_body,
        grid=(E // EPG,),
        in_specs=[
            pl.BlockSpec(memory_space=pltpu.SMEM),
            pl.BlockSpec((T, D), lambda e: (0, 0)),
            pl.BlockSpec((T, 128), lambda e: (0, 0)),
            pl.BlockSpec((EPG, FF, D), lambda e: (e, 0, 0)),
            pl.BlockSpec((EPG, FF, D), lambda e: (e, 0, 0)),
            pl.BlockSpec((EPG, D, FF), lambda e: (e, 0, 0)),
        ],
        out_specs=pl.BlockSpec((T, D), lambda e: (0, 0)),
        out_shape=jax.ShapeDtypeStruct((T, D), jnp.float32),
        compiler_params=pltpu.CompilerParams(
            dimension_semantics=("arbitrary",),
            vmem_limit_bytes=100_000_000,
        ),
    )(offs, xs, ws, w_gate_proj, w_up_proj, w_down_proj)


def _combine_body(ys_hbm, pos_hbm, out_hbm, pos_v, rows_v, sem):
    wid = jax.lax.axis_index("s") * 2 + jax.lax.axis_index("c")
    base = wid * TOK
    pltpu.sync_copy(pos_hbm.at[pl.ds(base, TOK)], pos_v)
    pltpu.async_copy(ys_hbm.at[pos_v], rows_v, sem).wait()
    pltpu.sync_copy(rows_v, out_hbm.at[pl.ds(base, TOK)])


def kernel(hidden_states, gate_w, w_gate_proj, w_up_proj, w_down_proj):
    bsz, seq_len, d = hidden_states.shape
    x = hidden_states.reshape(bsz * seq_len, d)
    logits, pos2d, wtop16, offs = _router(x, gate_w)
    dispatch, combine = _sc_kernels()
    pos = pos2d.reshape(-1)
    xs, ws = dispatch(x, pos, wtop16)
    ys = _mlp(offs, xs, ws, w_gate_proj, w_up_proj, w_down_proj)
    out = combine(ys, pos)
    return (out.reshape(bsz, seq_len, d), logits)


# speedup vs baseline: 1.0340x; 1.0340x over previous
import functools

import jax
import jax.numpy as jnp
from jax.experimental import pallas as pl
from jax.experimental.pallas import tpu as pltpu
from jax.experimental.pallas import tpu_sc as plsc

E = 64
D = 768
FF = 256
T = 2048
BT = 64
NW = 32
TOK = T // NW
OFFS_LEN = 80


def _router_body(x_ref, gw_ref, logits_ref, pos_ref, wtop_ref, offs_ref):
    x = x_ref[...]
    gw = gw_ref[...]
    logits = jax.lax.dot_general(
        x, gw, (((1,), (1,)), ((), ())), preferred_element_type=jnp.float32)
    logits_ref[...] = logits

    m = jnp.max(logits, axis=1, keepdims=True)
    s = jnp.sum(jnp.exp(logits - m), axis=1, keepdims=True)
    wtop_ref[...] = jnp.broadcast_to(1.0 / s, (T, 128))

    iota_e = jax.lax.broadcasted_iota(jnp.int32, (T, E), 1)
    eidx = jnp.min(jnp.where(logits == m, iota_e, E), axis=1, keepdims=True)
    onehot = (iota_e == eidx).astype(jnp.bfloat16)

    counts = jnp.sum(onehot.astype(jnp.float32), axis=0, keepdims=True)
    bi = jax.lax.broadcasted_iota(jnp.int32, (E, E), 0)
    bj = jax.lax.broadcasted_iota(jnp.int32, (E, E), 1)
    lt = (bi < bj).astype(jnp.float32)
    offs = jax.lax.dot_general(
        counts, lt, (((1,), (0,)), ((), ())), preferred_element_type=jnp.float32)

    blk = 256
    nb = T // blk
    ri = jax.lax.broadcasted_iota(jnp.int32, (blk, blk), 0)
    ci = jax.lax.broadcasted_iota(jnp.int32, (blk, blk), 1)
    g256 = (ci < ri).astype(jnp.bfloat16)
    ki = jax.lax.broadcasted_iota(jnp.int32, (nb, T), 0)
    ti = jax.lax.broadcasted_iota(jnp.int32, (nb, T), 1) // blk
    sel = (ki == ti).astype(jnp.bfloat16)
    bsum = jax.lax.dot_general(
        sel, onehot, (((1,), (0,)), ((), ())), preferred_element_type=jnp.float32)
    bi8 = jax.lax.broadcasted_iota(jnp.int32, (nb, nb), 0)
    bj8 = jax.lax.broadcasted_iota(jnp.int32, (nb, nb), 1)
    ltb = (bj8 < bi8).astype(jnp.float32)
    bpre = jax.lax.dot_general(
        ltb, bsum, (((1,), (0,)), ((), ())), preferred_element_type=jnp.float32)
    csum_parts = []
    for k in range(nb):
        local = jax.lax.dot_general(
            g256, onehot[k * blk:(k + 1) * blk, :], (((1,), (0,)), ((), ())),
            preferred_element_type=jnp.float32)
        csum_parts.append(local + bpre[k:k + 1, :])
    csum = jnp.concatenate(csum_parts, axis=0)

    posf = jnp.sum((csum + offs) * onehot.astype(jnp.float32),
                   axis=1, keepdims=True)
    pos_ref[...] = posf.astype(jnp.int32)

    offs_i = offs.astype(jnp.int32)
    offs_ref[...] = jnp.concatenate(
        [offs_i, jnp.full((1, OFFS_LEN - E), T, jnp.int32)], axis=1)


def _router(x, gate_w):
    return pl.pallas_call(
        _router_body,
        out_shape=(
            jax.ShapeDtypeStruct((T, E), jnp.float32),
            jax.ShapeDtypeStruct((T, 1), jnp.int32),
            jax.ShapeDtypeStruct((T, 128), jnp.float32),
            jax.ShapeDtypeStruct((1, OFFS_LEN), jnp.int32),
        ),
        compiler_params=pltpu.CompilerParams(vmem_limit_bytes=100_000_000),
    )(x, gate_w)


def _dispatch_body(x_hbm, pos_hbm, w16_hbm, xs_hbm, ws_hbm,
                   pos_v, rows_v, w16_v, sem_x, sem_w):
    wid = jax.lax.axis_index("s") * 2 + jax.lax.axis_index("c")
    base = wid * TOK
    pltpu.sync_copy(pos_hbm.at[pl.ds(base, TOK)], pos_v)
    pltpu.sync_copy(x_hbm.at[pl.ds(base, TOK)], rows_v)
    pltpu.sync_copy(w16_hbm.at[pl.ds(base, TOK)], w16_v)
    cp_x = pltpu.async_copy(rows_v, xs_hbm.at[pos_v], sem_x)
    cp_w = pltpu.async_copy(w16_v, ws_hbm.at[pos_v], sem_w)
    cp_x.wait()
    cp_w.wait()


@functools.cache
def _sc_kernels():
    mesh = plsc.VectorSubcoreMesh(core_axis_name="c", subcore_axis_name="s")
    dispatch = pl.kernel(
        _dispatch_body,
        out_type=[
            jax.ShapeDtypeStruct((T, D), jnp.float32),
            jax.ShapeDtypeStruct((T, 128), jnp.float32),
        ],
        mesh=mesh,
        scratch_types=[
            pltpu.VMEM((TOK,), jnp.int32),
            pltpu.VMEM((TOK, D), jnp.float32),
            pltpu.VMEM((TOK, 128), jnp.float32),
            pltpu.SemaphoreType.DMA,
            pltpu.SemaphoreType.DMA,
        ],
    )
    combine = pl.kernel(
        _combine_body,
        out_type=jax.ShapeDtypeStruct((T, D), jnp.float32),
        mesh=mesh,
        scratch_types=[
            pltpu.VMEM((TOK,), jnp.int32),
            pltpu.VMEM((TOK, D), jnp.float32),
            pltpu.SemaphoreType.DMA,
        ],
    )
    return dispatch, combine


EPG = 4


def _mlp_body(offs_ref, xs_ref, ws_ref, wg_ref, wu_ref, wd_ref, ys_ref):
    step = pl.program_id(0)

    @pl.when(step == 0)
    def _():
        ys_ref[...] = jnp.zeros_like(ys_ref)

    for j in range(EPG):
        e = step * EPG + j
        start = offs_ref[0, e]
        end = offs_ref[0, e + 1]
        s8 = (start // 8) * 8
        nb = (end - s8 + BT - 1) // BT
        wg = wg_ref[j].astype(jnp.bfloat16)
        wu = wu_ref[j].astype(jnp.bfloat16)
        wd = wd_ref[j].astype(jnp.bfloat16)

        def body(b, _, start=start, end=end, s8=s8, wg=wg, wu=wu, wd=wd):
            row0 = s8 + b * BT
            row0c = pl.multiple_of(jnp.minimum(row0, T - BT), 8)
            xb = xs_ref[pl.ds(row0c, BT), :].astype(jnp.bfloat16)
            g = jax.lax.dot_general(
                xb, wg, (((1,), (1,)), ((), ())),
                preferred_element_type=jnp.float32)
            u = jax.lax.dot_general(
                xb, wu, (((1,), (1,)), ((), ())),
                preferred_element_type=jnp.float32)
            h = g * (1.0 / (1.0 + jnp.exp(-g))) * u
            y = jax.lax.dot_general(
                h.astype(jnp.bfloat16), wd, (((1,), (1,)), ((), ())),
                preferred_element_type=jnp.float32)
            rows = row0c + jax.lax.broadcasted_iota(jnp.int32, (BT, 1), 0)
            mask = ((rows >= jnp.maximum(row0, start))
                    & (rows < jnp.minimum(row0 + BT, end))).astype(jnp.float32)
            wv = ws_ref[pl.ds(row0c, BT), 0:1]
            ys_ref[pl.ds(row0c, BT), :] += y * (wv * mask)
            return 0

        jax.lax.fori_loop(0, nb, body, 0)


def _mlp(offs, xs, ws, w_gate_proj, w_up_proj, w_down_proj):
    return pl.pallas_call(
        _mlp_body,
        grid=(E // EPG,),
        in_specs=[
            pl.BlockSpec(memory_space=pltpu.SMEM),
            pl.BlockSpec((T, D), lambda e: (0, 0)),
            pl.BlockSpec((T, 128), lambda e: (0, 0)),
            pl.BlockSpec((EPG, FF, D), lambda e: (e, 0, 0)),
            pl.BlockSpec((EPG, FF, D), lambda e: (e, 0, 0)),
            pl.BlockSpec((EPG, D, FF), lambda e: (e, 0, 0)),
        ],
        out_specs=pl.BlockSpec((T, D), lambda e: (0, 0)),
        out_shape=jax.ShapeDtypeStruct((T, D), jnp.float32),
        compiler_params=pltpu.CompilerParams(
            dimension_semantics=("arbitrary",),
            vmem_limit_bytes=100_000_000,
        ),
    )(offs, xs, ws, w_gate_proj, w_up_proj, w_down_proj)


def _combine_body(ys_hbm, pos_hbm, out_hbm, pos_v, rows_v, sem):
    wid = jax.lax.axis_index("s") * 2 + jax.lax.axis_index("c")
    base = wid * TOK
    pltpu.sync_copy(pos_hbm.at[pl.ds(base, TOK)], pos_v)
    pltpu.async_copy(ys_hbm.at[pos_v], rows_v, sem).wait()
    pltpu.sync_copy(rows_v, out_hbm.at[pl.ds(base, TOK)])


def kernel(hidden_states, gate_w, w_gate_proj, w_up_proj, w_down_proj):
    bsz, seq_len, d = hidden_states.shape
    x = hidden_states.reshape(bsz * seq_len, d)
    logits, pos2d, wtop16, offs = _router(x, gate_w)
    dispatch, combine = _sc_kernels()
    pos = pos2d.reshape(-1)
    xs, ws = dispatch(x, pos, wtop16)
    ys = _mlp(offs, xs, ws, w_gate_proj, w_up_proj, w_down_proj)
    out = combine(ys, pos)
    return (out.reshape(bsz, seq_len, d), logits)


# speedup vs baseline: 1.0465x; 1.0121x over previous
import functools

import jax
import jax.numpy as jnp
from jax.experimental import pallas as pl
from jax.experimental.pallas import tpu as pltpu
from jax.experimental.pallas import tpu_sc as plsc

E = 64
D = 768
FF = 256
T = 2048
BT = 64
NW = 32
TOK = T // NW
OFFS_LEN = 80


def _router_body(x_ref, gw_ref, logits_ref, pos_ref, wtop_ref, offs_ref):
    x = x_ref[...]
    gw = gw_ref[...]
    logits = jax.lax.dot_general(
        x, gw, (((1,), (1,)), ((), ())), preferred_element_type=jnp.float32)
    logits_ref[...] = logits

    m = jnp.max(logits, axis=1, keepdims=True)
    s = jnp.sum(jnp.exp(logits - m), axis=1, keepdims=True)
    wtop_ref[...] = jnp.broadcast_to(1.0 / s, (T, 128))

    iota_e = jax.lax.broadcasted_iota(jnp.int32, (T, E), 1)
    eidx = jnp.min(jnp.where(logits == m, iota_e, E), axis=1, keepdims=True)
    onehot = (iota_e == eidx).astype(jnp.bfloat16)

    counts = jnp.sum(onehot.astype(jnp.float32), axis=0, keepdims=True)
    bi = jax.lax.broadcasted_iota(jnp.int32, (E, E), 0)
    bj = jax.lax.broadcasted_iota(jnp.int32, (E, E), 1)
    lt = (bi < bj).astype(jnp.float32)
    offs = jax.lax.dot_general(
        counts, lt, (((1,), (0,)), ((), ())), preferred_element_type=jnp.float32)

    blk = 256
    nb = T // blk
    ri = jax.lax.broadcasted_iota(jnp.int32, (blk, blk), 0)
    ci = jax.lax.broadcasted_iota(jnp.int32, (blk, blk), 1)
    g256 = (ci < ri).astype(jnp.bfloat16)
    ki = jax.lax.broadcasted_iota(jnp.int32, (nb, T), 0)
    ti = jax.lax.broadcasted_iota(jnp.int32, (nb, T), 1) // blk
    sel = (ki == ti).astype(jnp.bfloat16)
    bsum = jax.lax.dot_general(
        sel, onehot, (((1,), (0,)), ((), ())), preferred_element_type=jnp.float32)
    bi8 = jax.lax.broadcasted_iota(jnp.int32, (nb, nb), 0)
    bj8 = jax.lax.broadcasted_iota(jnp.int32, (nb, nb), 1)
    ltb = (bj8 < bi8).astype(jnp.float32)
    bpre = jax.lax.dot_general(
        ltb, bsum, (((1,), (0,)), ((), ())), preferred_element_type=jnp.float32)
    csum_parts = []
    for k in range(nb):
        local = jax.lax.dot_general(
            g256, onehot[k * blk:(k + 1) * blk, :], (((1,), (0,)), ((), ())),
            preferred_element_type=jnp.float32)
        csum_parts.append(local + bpre[k:k + 1, :])
    csum = jnp.concatenate(csum_parts, axis=0)

    posf = jnp.sum((csum + offs) * onehot.astype(jnp.float32),
                   axis=1, keepdims=True)
    pos_ref[...] = posf.astype(jnp.int32)

    offs_i = offs.astype(jnp.int32)
    offs_ref[...] = jnp.concatenate(
        [offs_i, jnp.full((1, OFFS_LEN - E), T, jnp.int32)], axis=1)


def _router(x, gate_w):
    return pl.pallas_call(
        _router_body,
        out_shape=(
            jax.ShapeDtypeStruct((T, E), jnp.float32),
            jax.ShapeDtypeStruct((T, 1), jnp.int32),
            jax.ShapeDtypeStruct((T, 128), jnp.float32),
            jax.ShapeDtypeStruct((1, OFFS_LEN), jnp.int32),
        ),
        compiler_params=pltpu.CompilerParams(vmem_limit_bytes=100_000_000),
    )(x, gate_w)


def _dispatch_body(x_hbm, pos_hbm, w16_hbm, xs_hbm, ws_hbm,
                   pos_v, rows_v, w16_v, sem_x, sem_w):
    wid = jax.lax.axis_index("s") * 2 + jax.lax.axis_index("c")
    base = wid * TOK
    ld_p = pltpu.async_copy(pos_hbm.at[pl.ds(base, TOK)], pos_v, sem_x)
    ld_x = pltpu.async_copy(x_hbm.at[pl.ds(base, TOK)], rows_v, sem_w)
    ld_w = pltpu.async_copy(w16_hbm.at[pl.ds(base, TOK)], w16_v, sem_x)
    ld_p.wait()
    ld_x.wait()
    ld_w.wait()
    cp_x = pltpu.async_copy(rows_v, xs_hbm.at[pos_v], sem_x)
    cp_w = pltpu.async_copy(w16_v, ws_hbm.at[pos_v], sem_w)
    cp_x.wait()
    cp_w.wait()


@functools.cache
def _sc_kernels():
    mesh = plsc.VectorSubcoreMesh(core_axis_name="c", subcore_axis_name="s")
    dispatch = pl.kernel(
        _dispatch_body,
        out_type=[
            jax.ShapeDtypeStruct((T, D), jnp.float32),
            jax.ShapeDtypeStruct((T, 128), jnp.float32),
        ],
        mesh=mesh,
        scratch_types=[
            pltpu.VMEM((TOK,), jnp.int32),
            pltpu.VMEM((TOK, D), jnp.float32),
            pltpu.VMEM((TOK, 128), jnp.float32),
            pltpu.SemaphoreType.DMA,
            pltpu.SemaphoreType.DMA,
        ],
    )
    combine = pl.kernel(
        _combine_body,
        out_type=jax.ShapeDtypeStruct((T, D), jnp.float32),
        mesh=mesh,
        scratch_types=[
            pltpu.VMEM((TOK,), jnp.int32),
            pltpu.VMEM((TOK, D), jnp.float32),
            pltpu.SemaphoreType.DMA,
        ],
    )
    return dispatch, combine


EPG = 4


def _mlp_body(offs_ref, xs_ref, ws_ref, wg_ref, wu_ref, wd_ref, ys_ref):
    step = pl.program_id(0)

    @pl.when(step == 0)
    def _():
        ys_ref[...] = jnp.zeros_like(ys_ref)

    for j in range(EPG):
        e = step * EPG + j
        start = offs_ref[0, e]
        end = offs_ref[0, e + 1]
        s8 = (start // 8) * 8
        nb = (end - s8 + BT - 1) // BT
        wg = wg_ref[j].astype(jnp.bfloat16)
        wu = wu_ref[j].astype(jnp.bfloat16)
        wd = wd_ref[j].astype(jnp.bfloat16)

        def body(b, _, start=start, end=end, s8=s8, wg=wg, wu=wu, wd=wd):
            row0 = s8 + b * BT
            row0c = pl.multiple_of(jnp.minimum(row0, T - BT), 8)
            xb = xs_ref[pl.ds(row0c, BT), :].astype(jnp.bfloat16)
            g = jax.lax.dot_general(
                xb, wg, (((1,), (1,)), ((), ())),
                preferred_element_type=jnp.float32)
            u = jax.lax.dot_general(
                xb, wu, (((1,), (1,)), ((), ())),
                preferred_element_type=jnp.float32)
            h = g * (1.0 / (1.0 + jnp.exp(-g))) * u
            y = jax.lax.dot_general(
                h.astype(jnp.bfloat16), wd, (((1,), (1,)), ((), ())),
                preferred_element_type=jnp.float32)
            rows = row0c + jax.lax.broadcasted_iota(jnp.int32, (BT, 1), 0)
            mask = ((rows >= jnp.maximum(row0, start))
                    & (rows < jnp.minimum(row0 + BT, end))).astype(jnp.float32)
            wv = ws_ref[pl.ds(row0c, BT), 0:1]
            ys_ref[pl.ds(row0c, BT), :] += y * (wv * mask)
            return 0

        jax.lax.fori_loop(0, nb, body, 0)


def _mlp(offs, xs, ws, w_gate_proj, w_up_proj, w_down_proj):
    return pl.pallas_call(
        _mlp_body,
        grid=(E // EPG,),
        in_specs=[
            pl.BlockSpec(memory_space=pltpu.SMEM),
            pl.BlockSpec((T, D), lambda e: (0, 0)),
            pl.BlockSpec((T, 128), lambda e: (0, 0)),
            pl.BlockSpec((EPG, FF, D), lambda e: (e, 0, 0)),
            pl.BlockSpec((EPG, FF, D), lambda e: (e, 0, 0)),
            pl.BlockSpec((EPG, D, FF), lambda e: (e, 0, 0)),
        ],
        out_specs=pl.BlockSpec((T, D), lambda e: (0, 0)),
        out_shape=jax.ShapeDtypeStruct((T, D), jnp.float32),
        compiler_params=pltpu.CompilerParams(
            dimension_semantics=("arbitrary",),
            vmem_limit_bytes=100_000_000,
        ),
    )(offs, xs, ws, w_gate_proj, w_up_proj, w_down_proj)


def _combine_body(ys_hbm, pos_hbm, out_hbm, pos_v, rows_v, sem):
    wid = jax.lax.axis_index("s") * 2 + jax.lax.axis_index("c")
    base = wid * TOK
    pltpu.sync_copy(pos_hbm.at[pl.ds(base, TOK)], pos_v)
    pltpu.async_copy(ys_hbm.at[pos_v], rows_v, sem).wait()
    pltpu.sync_copy(rows_v, out_hbm.at[pl.ds(base, TOK)])


def kernel(hidden_states, gate_w, w_gate_proj, w_up_proj, w_down_proj):
    bsz, seq_len, d = hidden_states.shape
    x = hidden_states.reshape(bsz * seq_len, d)
    logits, pos2d, wtop16, offs = _router(x, gate_w)
    dispatch, combine = _sc_kernels()
    pos = pos2d.reshape(-1)
    xs, ws = dispatch(x, pos, wtop16)
    ys = _mlp(offs, xs, ws, w_gate_proj, w_up_proj, w_down_proj)
    out = combine(ys, pos)
    return (out.reshape(bsz, seq_len, d), logits)
